# baseline (speedup 1.0000x reference)
import jax
import jax.numpy as jnp
from jax.experimental import pallas as pl
from jax.experimental.pallas import tpu as pltpu

E = 64
D = 1024
FF = 1024
N = 2048
CAP = 40


def _moe_body(x_ref, gw_ref, wg_ref, wu_ref, wo_ref, y_ref,
              xg_scr, acc_scr, eidv_scr, eids_scr, idx_scr, cnt_scr, sem):
    e = pl.program_id(0)

    @pl.when(e == 0)
    def _prologue():
        y_ref[...] = jnp.zeros_like(y_ref)
        logits = jax.lax.dot_general(
            x_ref[...], gw_ref[...], (((1,), (1,)), ((), ())),
            preferred_element_type=jnp.float32)
        eidv_scr[...] = jnp.argmax(logits, axis=1).astype(
            jnp.int32).reshape(N // 128, 128)
        copy = pltpu.make_async_copy(eidv_scr, eids_scr, sem)
        copy.start()
        copy.wait()

        def zero_body(i, _):
            cnt_scr[i] = 0
            return 0
        jax.lax.fori_loop(0, E, zero_body, 0)

        def scan_body(t, _):
            ee = eids_scr[t // 128, t % 128]
            p = cnt_scr[ee]
            idx_scr[ee, jnp.minimum(p, CAP)] = t
            cnt_scr[ee] = p + 1
            return 0
        jax.lax.fori_loop(0, N, scan_body, 0, unroll=16)

    cnt = jnp.minimum(cnt_scr[e], CAP)

    def gbody(c, _):
        src = jnp.where(c < cnt, idx_scr[e, c], 0)
        xg_scr[pl.ds(c, 1), :] = x_ref[pl.ds(src, 1), :]
        return 0
    jax.lax.fori_loop(0, CAP, gbody, 0, unroll=CAP)

    xg = xg_scr[...]
    g = jax.lax.dot_general(xg, wg_ref[0], (((1,), (1,)), ((), ())),
                            preferred_element_type=jnp.float32)
    u = jax.lax.dot_general(xg, wu_ref[0], (((1,), (1,)), ((), ())),
                            preferred_element_type=jnp.float32)
    h = (g * jax.nn.sigmoid(g)) * u
    part = jax.lax.dot_general(h, wo_ref[0], (((1,), (1,)), ((), ())),
                               preferred_element_type=jnp.float32)
    acc_scr[...] = part

    def sbody(c, _):
        dst = jnp.where(c < cnt, idx_scr[e, c], N)
        y_ref[pl.ds(dst, 1), :] = acc_scr[pl.ds(c, 1), :]
        return 0
    jax.lax.fori_loop(0, CAP, sbody, 0, unroll=CAP)


def kernel(x, gate_w, wi_gate, wi_up, wo):
    B, S, D_ = x.shape
    xf = x.reshape(N, D)

    ypad = pl.pallas_call(
        _moe_body,
        grid=(E,),
        in_specs=[
            pl.BlockSpec((N, D), lambda e: (0, 0)),
            pl.BlockSpec((E, D), lambda e: (0, 0)),
            pl.BlockSpec((1, FF, D), lambda e: (e, 0, 0)),
            pl.BlockSpec((1, FF, D), lambda e: (e, 0, 0)),
            pl.BlockSpec((1, D, FF), lambda e: (e, 0, 0)),
        ],
        out_specs=pl.BlockSpec((N + 8, D), lambda e: (0, 0)),
        out_shape=jax.ShapeDtypeStruct((N + 8, D), jnp.float32),
        scratch_shapes=[
            pltpu.VMEM((CAP, D), jnp.float32),
            pltpu.VMEM((CAP, D), jnp.float32),
            pltpu.VMEM((N // 128, 128), jnp.int32),
            pltpu.SMEM((N // 128, 128), jnp.int32),
            pltpu.SMEM((E, CAP + 1), jnp.int32),
            pltpu.SMEM((E,), jnp.int32),
            pltpu.SemaphoreType.DMA,
        ],
    )(xf, gate_w, wi_gate, wi_up, wo)

    return ypad[:N].reshape(B, S, D_)
